# baseline (speedup 1.0000x reference)
import functools

import jax
import jax.numpy as jnp
from jax.experimental import pallas as pl


def _loss_body(t_ref, x_ref, o_ref, *, ncols):
    rows = x_ref.shape[0]
    cols = jax.lax.broadcasted_iota(jnp.int32, (rows, ncols), 1)
    logs = jnp.log(x_ref[...])
    masked = jnp.where(cols == t_ref[...], 0.0, logs)
    o_ref[...] = jnp.sum(masked, axis=1, keepdims=True) * (-1.0 / ncols)


def kernel(pred, target):
    B, C = pred.shape
    BR = 16
    t2 = target.astype(jnp.int32).reshape(B, 1)
    out = pl.pallas_call(
        functools.partial(_loss_body, ncols=C),
        grid=(B // BR,),
        in_specs=[
            pl.BlockSpec((BR, 1), lambda i: (i, 0)),
            pl.BlockSpec((BR, C), lambda i: (i, 0)),
        ],
        out_specs=pl.BlockSpec((BR, 1), lambda i: (i, 0)),
        out_shape=jax.ShapeDtypeStruct((B, 1), jnp.float32),
    )(t2, pred)
    return out[:, 0]


# speedup vs baseline: 1.0020x; 1.0020x over previous
import functools

import jax
import jax.numpy as jnp
from jax.experimental import pallas as pl
from jax.experimental.pallas import tpu as pltpu


def _loss_body(t_ref, x_ref, o_ref, *, ncols):
    rows = x_ref.shape[0]
    cols = jax.lax.broadcasted_iota(jnp.int32, (rows, ncols), 1)
    logs = jnp.log(x_ref[...])
    masked = jnp.where(cols == t_ref[...], 0.0, logs)
    o_ref[...] = jnp.sum(masked, axis=1, keepdims=True) * (-1.0 / ncols)


def kernel(pred, target):
    B, C = pred.shape
    BR = 16
    t2 = target.astype(jnp.int32).reshape(B, 1)
    out = pl.pallas_call(
        functools.partial(_loss_body, ncols=C),
        grid=(B // BR,),
        in_specs=[
            pl.BlockSpec((BR, 1), lambda i: (i, 0)),
            pl.BlockSpec((BR, C), lambda i: (i, 0)),
        ],
        out_specs=pl.BlockSpec((BR, 1), lambda i: (i, 0)),
        out_shape=jax.ShapeDtypeStruct((B, 1), jnp.float32),
        compiler_params=pltpu.CompilerParams(
            dimension_semantics=("parallel",)),
    )(t2, pred)
    return out[:, 0]
